# Initial kernel scaffold; baseline (speedup 1.0000x reference)
import jax
import jax.numpy as jnp
from jax.experimental import pallas as pl


def kernel(species, edge_idx, edge_diff, edge_dist, W_chem, Wr1_0, Wr2_0, Wlin_0, Wsc_0, Wr1_1, Wr2_1, Wlin_1, Wsc_1, W_ro1, W_ro2):
    raise NotImplementedError("write your pallas kernel here")


# speedup vs baseline: 16.8547x; 16.8547x over previous
import functools

import jax
import jax.numpy as jnp
from jax import lax
from jax.experimental import pallas as pl
from jax.experimental.pallas import tpu as pltpu
from jax.experimental.pallas import tpu_sc as plsc

N = 50000
E = 800000
NUM_ELEM = 64
F = 16
SH = 4
CUTOFF = 5.0
NB = 8

NPAD = 51200
ROWS_PER_TILE = NPAD // 16
CHUNK = 128

G_PER_W = E // 32
G_FULL = G_PER_W // CHUNK
G_TAIL = G_PER_W - G_FULL * CHUNK
S_PER_T = E // 16
S_FULL = S_PER_T // CHUNK
S_TAIL = S_PER_T - S_FULL * CHUNK

INV_NORM = 1.0 / ((E / N) ** 0.5)
SQRT3 = 3.0 ** 0.5

def _gather_body(h_hbm, src_hbm, out_hbm, idx_v, rows_v, idx_t, rows_t, sem):
    wid = lax.axis_index("s") * 2 + lax.axis_index("c")
    base_w = wid * G_PER_W

    def body(i, _):
        base = base_w + i * CHUNK
        pltpu.sync_copy(src_hbm.at[pl.ds(base, CHUNK)], idx_v)
        pltpu.async_copy(h_hbm.at[idx_v], rows_v, sem).wait()
        pltpu.sync_copy(rows_v, out_hbm.at[pl.ds(base, CHUNK)])
        return _

    lax.fori_loop(0, G_FULL, body, None)
    base = base_w + G_FULL * CHUNK
    pltpu.sync_copy(src_hbm.at[pl.ds(base, G_TAIL)], idx_t)
    pltpu.async_copy(h_hbm.at[idx_t], rows_t, sem).wait()
    pltpu.sync_copy(rows_t, out_hbm.at[pl.ds(base, G_TAIL)])


@functools.lru_cache(maxsize=None)
def _gather_sc():
    mesh = plsc.VectorSubcoreMesh(core_axis_name="c", subcore_axis_name="s")
    return pl.kernel(
        _gather_body,
        mesh=mesh,
        out_type=jax.ShapeDtypeStruct((E, 64), jnp.float32),
        scratch_types=[
            pltpu.VMEM((CHUNK,), jnp.int32),
            pltpu.VMEM((CHUNK, 64), jnp.float32),
            pltpu.VMEM((G_TAIL,), jnp.int32),
            pltpu.VMEM((G_TAIL, 64), jnp.float32),
            pltpu.SemaphoreType.DMA,
        ],
        compiler_params=pltpu.CompilerParams(use_tc_tiling_on_sc=False),
    )


def _scatter_body(msg_hbm, dst_hbm, zeros_hbm, agg_hbm, idx_v, msg_v, idx_t,
                  msg_t, shared):
    cid = lax.axis_index("c")
    sid = lax.axis_index("s")
    fbase = cid * 32
    rbase = sid * ROWS_PER_TILE
    pltpu.sync_copy(zeros_hbm, shared.at[pl.ds(rbase, ROWS_PER_TILE)])
    plsc.subcore_barrier()

    tile_base = sid * S_PER_T

    def body(i, _):
        base = tile_base + i * CHUNK
        pltpu.sync_copy(dst_hbm.at[pl.ds(base, CHUNK)], idx_v)
        pltpu.sync_copy(msg_hbm.at[pl.ds(base, CHUNK), pl.ds(fbase, 32)],
                        msg_v)
        pltpu.sync_copy(msg_v, shared.at[idx_v], add=True)
        return _

    lax.fori_loop(0, S_FULL, body, None)
    base = tile_base + S_FULL * CHUNK
    pltpu.sync_copy(dst_hbm.at[pl.ds(base, S_TAIL)], idx_t)
    pltpu.sync_copy(msg_hbm.at[pl.ds(base, S_TAIL), pl.ds(fbase, 32)], msg_t)
    pltpu.sync_copy(msg_t, shared.at[idx_t], add=True)

    plsc.subcore_barrier()
    pltpu.sync_copy(shared.at[pl.ds(rbase, ROWS_PER_TILE)],
                    agg_hbm.at[pl.ds(rbase, ROWS_PER_TILE),
                               pl.ds(fbase, 32)])


@functools.lru_cache(maxsize=None)
def _scatter_sc():
    mesh = plsc.VectorSubcoreMesh(core_axis_name="c", subcore_axis_name="s")
    return pl.kernel(
        _scatter_body,
        mesh=mesh,
        out_type=jax.ShapeDtypeStruct((NPAD, 64), jnp.float32),
        scratch_types=[
            pltpu.VMEM((CHUNK,), jnp.int32),
            pltpu.VMEM((CHUNK, 32), jnp.float32),
            pltpu.VMEM((S_TAIL,), jnp.int32),
            pltpu.VMEM((S_TAIL, 32), jnp.float32),
            pltpu.VMEM_SHARED((NPAD, 32), jnp.float32),
        ],
        compiler_params=pltpu.CompilerParams(use_tc_tiling_on_sc=False),
    )


BE = 1000


def _msg_body(hs_ref, d_ref, dx_ref, dy_ref, dz_ref, wr1_ref, wr2_ref,
              out_ref):
    d = d_ref[...]
    n = lax.broadcasted_iota(jnp.int32, (1, NB), 1).astype(jnp.float32) + 1.0
    basis = jnp.sqrt(2.0 / CUTOFF) * jnp.sin(
        n * (jnp.pi / CUTOFF) * d) / (d + 1e-9)
    t = basis @ wr1_ref[...]
    t = t * jax.nn.sigmoid(t)
    R = t @ wr2_ref[...]
    x = d * (1.0 / CUTOFF)
    x2 = x * x
    x4 = x2 * x2
    p = 6.0
    env = (1.0 - ((p + 1.0) * (p + 2.0) / 2.0) * (x4 * x2)
           + p * (p + 2.0) * (x4 * x2 * x)
           - (p * (p + 1.0) / 2.0) * (x4 * x4))
    env = env * (x < 1.0).astype(jnp.float32)
    R2 = R * (0.5 * env)
    dx = dx_ref[...]
    dy = dy_ref[...]
    dz = dz_ref[...]
    inv_r = 1.0 / (jnp.sqrt(dx * dx + dy * dy + dz * dz) + 1e-9)
    ones = jnp.ones((BE, F), jnp.float32)
    sh_exp = jnp.concatenate(
        [ones,
         (SQRT3 * dx * inv_r) * ones,
         (SQRT3 * dy * inv_r) * ones,
         (SQRT3 * dz * inv_r) * ones], axis=1)
    h = hs_ref[...]
    h0 = h[:, :F]
    h0cat = jnp.concatenate([h0, h0, h0, h0], axis=1)
    R4 = jnp.concatenate([R2, R2, R2, R2], axis=1)
    out_ref[...] = R4 * (sh_exp * h0cat + h)


def _msg_tc(h_src, dist, dx, dy, dz, Wr1, Wr2):
    grid = E // BE
    eb = lambda w: pl.BlockSpec((BE, w), lambda i: (i, 0))
    full = lambda a, b: pl.BlockSpec((a, b), lambda i: (0, 0))
    return pl.pallas_call(
        _msg_body,
        grid=(grid,),
        in_specs=[eb(64), eb(1), eb(1), eb(1), eb(1), full(NB, F),
                  full(F, F)],
        out_specs=eb(64),
        out_shape=jax.ShapeDtypeStruct((E, 64), jnp.float32),
    )(h_src, dist, dx, dy, dz, Wr1, Wr2)


BN = 1024


def _node_body(agg_ref, hp_ref, sp_ref, wsc_ref, wlin_ref, out_ref):
    onehot = (sp_ref[...] ==
              lax.broadcasted_iota(jnp.int32, (1, NUM_ELEM), 1)
              ).astype(jnp.float32)
    Wn = jnp.dot(onehot, wsc_ref[...],
                 preferred_element_type=jnp.float32)
    agg = agg_ref[...]
    hp = hp_ref[...]
    wlin = wlin_ref[...]
    hn = []
    for c in range(SH):
        agg_c = agg[:, c * F:(c + 1) * F] * INV_NORM
        lin_c = jnp.dot(agg_c, wlin, preferred_element_type=jnp.float32)
        h_c = hp[:, c * F:(c + 1) * F]
        sc_c = jnp.zeros((BN, F), jnp.float32)
        for f in range(F):
            sc_c = sc_c + h_c[:, f:f + 1] * Wn[:, f * F:(f + 1) * F]
        hn.append(lin_c + sc_c)
    g = jax.nn.sigmoid(hn[0])
    out_ref[...] = jnp.concatenate([hn[0] * g, hn[1] * g, hn[2] * g,
                                    hn[3] * g], axis=1)


def _node_tc(agg, h_prev, spec2d, Wsc_flat, Wlin):
    grid = NPAD // BN
    nb = lambda w: pl.BlockSpec((BN, w), lambda i: (i, 0))
    full = lambda a, b: pl.BlockSpec((a, b), lambda i: (0, 0))
    return pl.pallas_call(
        _node_body,
        grid=(grid,),
        in_specs=[nb(64), nb(64), nb(1), full(NUM_ELEM, 256), full(F, F)],
        out_specs=nb(64),
        out_shape=jax.ShapeDtypeStruct((NPAD, 64), jnp.float32),
    )(agg, h_prev, spec2d, Wsc_flat, Wlin)


def _embed_body(sp_ref, wch_ref, out_ref):
    onehot = (sp_ref[...] ==
              lax.broadcasted_iota(jnp.int32, (1, NUM_ELEM), 1)
              ).astype(jnp.float32)
    x0 = jnp.dot(onehot, wch_ref[...], preferred_element_type=jnp.float32)
    out_ref[...] = jnp.concatenate(
        [x0, jnp.zeros((BN, 48), jnp.float32)], axis=1)


def _embed_tc(spec2d, W_chem):
    grid = NPAD // BN
    return pl.pallas_call(
        _embed_body,
        grid=(grid,),
        in_specs=[pl.BlockSpec((BN, 1), lambda i: (i, 0)),
                  pl.BlockSpec((NUM_ELEM, F), lambda i: (0, 0))],
        out_specs=pl.BlockSpec((BN, 64), lambda i: (i, 0)),
        out_shape=jax.ShapeDtypeStruct((NPAD, 64), jnp.float32),
    )(spec2d, W_chem)


def _readout_body(h_ref, w1_ref, w2_ref, out_ref):
    s = h_ref[:, :F]
    t = jnp.dot(s, w1_ref[...], preferred_element_type=jnp.float32)
    t = t * jax.nn.sigmoid(t)
    out_ref[...] = jnp.dot(t, w2_ref[...],
                           preferred_element_type=jnp.float32)


def _readout_tc(h, W_ro1, W_ro2):
    grid = NPAD // BN
    return pl.pallas_call(
        _readout_body,
        grid=(grid,),
        in_specs=[pl.BlockSpec((BN, 64), lambda i: (i, 0)),
                  pl.BlockSpec((F, 16), lambda i: (0, 0)),
                  pl.BlockSpec((16, 1), lambda i: (0, 0))],
        out_specs=pl.BlockSpec((BN, 1), lambda i: (i, 0)),
        out_shape=jax.ShapeDtypeStruct((NPAD, 1), jnp.float32),
    )(h, W_ro1, W_ro2)


def kernel(species, edge_idx, edge_diff, edge_dist, W_chem, Wr1_0, Wr2_0,
           Wlin_0, Wsc_0, Wr1_1, Wr2_1, Wlin_1, Wsc_1, W_ro1, W_ro2):
    src = edge_idx[0].astype(jnp.int32)
    dst = edge_idx[1].astype(jnp.int32)
    dist = edge_dist.reshape(E, 1)
    dx = edge_diff[:, 0:1]
    dy = edge_diff[:, 1:2]
    dz = edge_diff[:, 2:3]
    spec2d = jnp.pad(species.astype(jnp.int32), (0, NPAD - N)).reshape(
        NPAD, 1)
    zeros_tile = jnp.zeros((ROWS_PER_TILE, 32), jnp.float32)

    h = _embed_tc(spec2d, W_chem)
    for Wr1, Wr2, Wlin, Wsc in ((Wr1_0, Wr2_0, Wlin_0, Wsc_0),
                                (Wr1_1, Wr2_1, Wlin_1, Wsc_1)):
        h_src = _gather_sc()(h, src)
        msg = _msg_tc(h_src, dist, dx, dy, dz, Wr1, Wr2)
        agg = _scatter_sc()(msg, dst, zeros_tile)
        h = _node_tc(agg, h, spec2d, Wsc.reshape(NUM_ELEM, 256), Wlin)
    e = _readout_tc(h, W_ro1, W_ro2)
    return e[:N, 0]


# speedup vs baseline: 18.0091x; 1.0685x over previous
import functools

import jax
import jax.numpy as jnp
from jax import lax
from jax.experimental import pallas as pl
from jax.experimental.pallas import tpu as pltpu
from jax.experimental.pallas import tpu_sc as plsc

N = 50000
E = 800000
NUM_ELEM = 64
F = 16
SH = 4
CUTOFF = 5.0
NB = 8

NPAD = 51200
ROWS_PER_TILE = NPAD // 16
CHUNK = 128

NCHUNK = E // CHUNK
G_CH = 196
G_GRP = G_CH // 4
GPAD_ROWS = 32 * G_CH
EPAD = GPAD_ROWS * CHUNK
S_CH = 391
SPAD_ROWS = 6256

INV_NORM = 1.0 / ((E / N) ** 0.5)
SQRT3 = 3.0 ** 0.5

def _gather_body(h_hbm, src2_hbm, out_hbm, idx_v, rows_v, sem):
    wid = lax.axis_index("s") * 2 + lax.axis_index("c")
    start = wid * (G_CH - 1) + jnp.minimum(wid, 10)
    pltpu.sync_copy(src2_hbm.at[pl.ds(start, G_CH)], idx_v)

    def body(g, _):
        c0 = g * 4
        ds = [
            pltpu.async_copy(h_hbm.at[idx_v.at[c0 + j]],
                             rows_v.at[pl.ds(j * CHUNK, CHUNK)], sem)
            for j in range(4)
        ]
        for d in ds:
            d.wait()
        pltpu.sync_copy(rows_v,
                        out_hbm.at[pl.ds((start + c0) * CHUNK, 4 * CHUNK)])
        return _

    lax.fori_loop(0, G_GRP, body, None)


@functools.lru_cache(maxsize=None)
def _gather_sc():
    mesh = plsc.VectorSubcoreMesh(core_axis_name="c", subcore_axis_name="s")
    return pl.kernel(
        _gather_body,
        mesh=mesh,
        out_type=jax.ShapeDtypeStruct((EPAD, 64), jnp.float32),
        scratch_types=[
            pltpu.VMEM((G_CH, CHUNK), jnp.int32),
            pltpu.VMEM((4 * CHUNK, 64), jnp.float32),
            pltpu.SemaphoreType.DMA,
        ],
        compiler_params=pltpu.CompilerParams(use_tc_tiling_on_sc=False),
    )


def _scatter_body(msg_hbm, dst2_hbm, zeros_hbm, agg_hbm, idx_v, msg_v,
                  shared, sem):
    cid = lax.axis_index("c")
    sid = lax.axis_index("s")
    fbase = cid * 32
    rbase = sid * ROWS_PER_TILE
    pltpu.sync_copy(zeros_hbm, shared.at[pl.ds(rbase, ROWS_PER_TILE)])
    plsc.subcore_barrier()

    start = sid * (S_CH - 1) + jnp.minimum(sid, 10)

    def add_chunks(k):
        ds = [
            pltpu.async_copy(msg_v.at[pl.ds(j * CHUNK, CHUNK)],
                             shared.at[idx_v.at[j]], sem, add=True)
            for j in range(k)
        ]
        for d in ds:
            d.wait()

    def body(g, _):
        c0 = g * 4
        pltpu.sync_copy(dst2_hbm.at[pl.ds(start + c0, 4)], idx_v)
        pltpu.sync_copy(
            msg_hbm.at[pl.ds((start + c0) * CHUNK, 4 * CHUNK),
                       pl.ds(fbase, 32)], msg_v)
        add_chunks(4)
        return _

    lax.fori_loop(0, 97, body, None)
    pltpu.sync_copy(dst2_hbm.at[pl.ds(start + 388, 2)],
                    idx_v.at[pl.ds(0, 2)])
    pltpu.sync_copy(
        msg_hbm.at[pl.ds((start + 388) * CHUNK, 2 * CHUNK),
                   pl.ds(fbase, 32)], msg_v.at[pl.ds(0, 2 * CHUNK)])
    add_chunks(2)

    @pl.when(sid < 10)
    def _extra():
        pltpu.sync_copy(dst2_hbm.at[pl.ds(start + 390, 1)],
                        idx_v.at[pl.ds(0, 1)])
        pltpu.sync_copy(
            msg_hbm.at[pl.ds((start + 390) * CHUNK, CHUNK),
                       pl.ds(fbase, 32)], msg_v.at[pl.ds(0, CHUNK)])
        pltpu.async_copy(msg_v.at[pl.ds(0, CHUNK)],
                         shared.at[idx_v.at[0]], sem, add=True).wait()

    plsc.subcore_barrier()
    pltpu.sync_copy(shared.at[pl.ds(rbase, ROWS_PER_TILE)],
                    agg_hbm.at[pl.ds(rbase, ROWS_PER_TILE),
                               pl.ds(fbase, 32)])


@functools.lru_cache(maxsize=None)
def _scatter_sc():
    mesh = plsc.VectorSubcoreMesh(core_axis_name="c", subcore_axis_name="s")
    return pl.kernel(
        _scatter_body,
        mesh=mesh,
        out_type=jax.ShapeDtypeStruct((NPAD, 64), jnp.float32),
        scratch_types=[
            pltpu.VMEM((4, CHUNK), jnp.int32),
            pltpu.VMEM((4 * CHUNK, 32), jnp.float32),
            pltpu.VMEM_SHARED((NPAD, 32), jnp.float32),
            pltpu.SemaphoreType.DMA,
        ],
        compiler_params=pltpu.CompilerParams(use_tc_tiling_on_sc=False),
    )


BE = 1000


def _msg_body(hs_ref, d_ref, dx_ref, dy_ref, dz_ref, wr1_ref, wr2_ref,
              out_ref):
    d = d_ref[...]
    n = lax.broadcasted_iota(jnp.int32, (1, NB), 1).astype(jnp.float32) + 1.0
    basis = jnp.sqrt(2.0 / CUTOFF) * jnp.sin(
        n * (jnp.pi / CUTOFF) * d) / (d + 1e-9)
    t = basis @ wr1_ref[...]
    t = t * jax.nn.sigmoid(t)
    R = t @ wr2_ref[...]
    x = d * (1.0 / CUTOFF)
    x2 = x * x
    x4 = x2 * x2
    p = 6.0
    env = (1.0 - ((p + 1.0) * (p + 2.0) / 2.0) * (x4 * x2)
           + p * (p + 2.0) * (x4 * x2 * x)
           - (p * (p + 1.0) / 2.0) * (x4 * x4))
    env = env * (x < 1.0).astype(jnp.float32)
    R2 = R * (0.5 * env)
    dx = dx_ref[...]
    dy = dy_ref[...]
    dz = dz_ref[...]
    inv_r = 1.0 / (jnp.sqrt(dx * dx + dy * dy + dz * dz) + 1e-9)
    ones = jnp.ones((BE, F), jnp.float32)
    sh_exp = jnp.concatenate(
        [ones,
         (SQRT3 * dx * inv_r) * ones,
         (SQRT3 * dy * inv_r) * ones,
         (SQRT3 * dz * inv_r) * ones], axis=1)
    h = hs_ref[...]
    h0 = h[:, :F]
    h0cat = jnp.concatenate([h0, h0, h0, h0], axis=1)
    R4 = jnp.concatenate([R2, R2, R2, R2], axis=1)
    out_ref[...] = R4 * (sh_exp * h0cat + h)


def _msg_tc(h_src, dist, dx, dy, dz, Wr1, Wr2):
    grid = E // BE
    eb = lambda w: pl.BlockSpec((BE, w), lambda i: (i, 0))
    full = lambda a, b: pl.BlockSpec((a, b), lambda i: (0, 0))
    return pl.pallas_call(
        _msg_body,
        grid=(grid,),
        in_specs=[eb(64), eb(1), eb(1), eb(1), eb(1), full(NB, F),
                  full(F, F)],
        out_specs=eb(64),
        out_shape=jax.ShapeDtypeStruct((E, 64), jnp.float32),
    )(h_src, dist, dx, dy, dz, Wr1, Wr2)


BN = 1024


def _node_body(agg_ref, hp_ref, sp_ref, wsc_ref, wlin_ref, out_ref):
    onehot = (sp_ref[...] ==
              lax.broadcasted_iota(jnp.int32, (1, NUM_ELEM), 1)
              ).astype(jnp.float32)
    Wn = jnp.dot(onehot, wsc_ref[...],
                 preferred_element_type=jnp.float32)
    agg = agg_ref[...]
    hp = hp_ref[...]
    wlin = wlin_ref[...]
    hn = []
    for c in range(SH):
        agg_c = agg[:, c * F:(c + 1) * F] * INV_NORM
        lin_c = jnp.dot(agg_c, wlin, preferred_element_type=jnp.float32)
        h_c = hp[:, c * F:(c + 1) * F]
        sc_c = jnp.zeros((BN, F), jnp.float32)
        for f in range(F):
            sc_c = sc_c + h_c[:, f:f + 1] * Wn[:, f * F:(f + 1) * F]
        hn.append(lin_c + sc_c)
    g = jax.nn.sigmoid(hn[0])
    out_ref[...] = jnp.concatenate([hn[0] * g, hn[1] * g, hn[2] * g,
                                    hn[3] * g], axis=1)


def _node_tc(agg, h_prev, spec2d, Wsc_flat, Wlin):
    grid = NPAD // BN
    nb = lambda w: pl.BlockSpec((BN, w), lambda i: (i, 0))
    full = lambda a, b: pl.BlockSpec((a, b), lambda i: (0, 0))
    return pl.pallas_call(
        _node_body,
        grid=(grid,),
        in_specs=[nb(64), nb(64), nb(1), full(NUM_ELEM, 256), full(F, F)],
        out_specs=nb(64),
        out_shape=jax.ShapeDtypeStruct((NPAD, 64), jnp.float32),
    )(agg, h_prev, spec2d, Wsc_flat, Wlin)


def _embed_body(sp_ref, wch_ref, out_ref):
    onehot = (sp_ref[...] ==
              lax.broadcasted_iota(jnp.int32, (1, NUM_ELEM), 1)
              ).astype(jnp.float32)
    x0 = jnp.dot(onehot, wch_ref[...], preferred_element_type=jnp.float32)
    out_ref[...] = jnp.concatenate(
        [x0, jnp.zeros((BN, 48), jnp.float32)], axis=1)


def _embed_tc(spec2d, W_chem):
    grid = NPAD // BN
    return pl.pallas_call(
        _embed_body,
        grid=(grid,),
        in_specs=[pl.BlockSpec((BN, 1), lambda i: (i, 0)),
                  pl.BlockSpec((NUM_ELEM, F), lambda i: (0, 0))],
        out_specs=pl.BlockSpec((BN, 64), lambda i: (i, 0)),
        out_shape=jax.ShapeDtypeStruct((NPAD, 64), jnp.float32),
    )(spec2d, W_chem)


def _readout_body(h_ref, w1_ref, w2_ref, out_ref):
    s = h_ref[:, :F]
    t = jnp.dot(s, w1_ref[...], preferred_element_type=jnp.float32)
    t = t * jax.nn.sigmoid(t)
    out_ref[...] = jnp.dot(t, w2_ref[...],
                           preferred_element_type=jnp.float32)


def _readout_tc(h, W_ro1, W_ro2):
    grid = NPAD // BN
    return pl.pallas_call(
        _readout_body,
        grid=(grid,),
        in_specs=[pl.BlockSpec((BN, 64), lambda i: (i, 0)),
                  pl.BlockSpec((F, 16), lambda i: (0, 0)),
                  pl.BlockSpec((16, 1), lambda i: (0, 0))],
        out_specs=pl.BlockSpec((BN, 1), lambda i: (i, 0)),
        out_shape=jax.ShapeDtypeStruct((NPAD, 1), jnp.float32),
    )(h, W_ro1, W_ro2)


def kernel(species, edge_idx, edge_diff, edge_dist, W_chem, Wr1_0, Wr2_0,
           Wlin_0, Wsc_0, Wr1_1, Wr2_1, Wlin_1, Wsc_1, W_ro1, W_ro2):
    src2 = jnp.pad(edge_idx[0].astype(jnp.int32),
                   (0, GPAD_ROWS * CHUNK - E)).reshape(GPAD_ROWS, CHUNK)
    dst2 = jnp.pad(edge_idx[1].astype(jnp.int32),
                   (0, SPAD_ROWS * CHUNK - E)).reshape(SPAD_ROWS, CHUNK)
    dist = edge_dist.reshape(E, 1)
    dx = edge_diff[:, 0:1]
    dy = edge_diff[:, 1:2]
    dz = edge_diff[:, 2:3]
    spec2d = jnp.pad(species.astype(jnp.int32), (0, NPAD - N)).reshape(
        NPAD, 1)
    zeros_tile = jnp.zeros((ROWS_PER_TILE, 32), jnp.float32)

    h = _embed_tc(spec2d, W_chem)
    for Wr1, Wr2, Wlin, Wsc in ((Wr1_0, Wr2_0, Wlin_0, Wsc_0),
                                (Wr1_1, Wr2_1, Wlin_1, Wsc_1)):
        h_src = _gather_sc()(h, src2)
        msg = _msg_tc(h_src, dist, dx, dy, dz, Wr1, Wr2)
        agg = _scatter_sc()(msg, dst2, zeros_tile)
        h = _node_tc(agg, h, spec2d, Wsc.reshape(NUM_ELEM, 256), Wlin)
    e = _readout_tc(h, W_ro1, W_ro2)
    return e[:N, 0]
